# baseline (speedup 1.0000x reference)
import jax
import jax.numpy as jnp
from jax.experimental import pallas as pl
from jax.experimental.pallas import tpu as pltpu

N0, N1, N2 = 10000, 160000, 50000
BATCH = 128
D = 128
C = 16


def _head_body(pooled_ref, w_ref, b_ref, out_ref):
    logits = jnp.dot(pooled_ref[...], w_ref[...],
                     preferred_element_type=jnp.float32) + b_ref[...]
    mx = jnp.max(logits, axis=-1, keepdims=True)
    sh = logits - mx
    lse = jnp.log(jnp.sum(jnp.exp(sh), axis=-1, keepdims=True))
    out_ref[...] = sh - lse


def _head(pooled, W, b):
    return pl.pallas_call(
        _head_body,
        out_shape=jax.ShapeDtypeStruct((BATCH, C), jnp.float32),
    )(pooled, W, b.reshape(1, C))


def kernel(x0, x1, x2, up_index0, up_index1, b1_src, b1_dst, b2_src, b2_dst,
           batch0, batch1, batch2, W, b):
    xs = [x0, x1, x2]
    for _ in range(2):
        up0 = jax.ops.segment_sum(xs[0][up_index0[0]], up_index0[1], num_segments=N0)
        up1 = jax.ops.segment_sum(xs[1][up_index1[0]], up_index1[1], num_segments=N1)
        bd1 = jax.ops.segment_sum(xs[0][b1_src], b1_dst, num_segments=N1)
        bd2 = jax.ops.segment_sum(xs[1][b2_src], b2_dst, num_segments=N2)
        xs = [xs[0] + up0, xs[1] + up1 + bd1, xs[2] + bd2]
    p0 = jax.ops.segment_sum(xs[0], batch0, num_segments=BATCH)
    p1 = jax.ops.segment_sum(xs[1], batch1, num_segments=BATCH)
    p2 = jax.ops.segment_sum(xs[2], batch2, num_segments=BATCH)
    pooled = p0 + p1 + p2
    return _head(pooled, W, b)


# speedup vs baseline: 1.3984x; 1.3984x over previous
import functools

import jax
import jax.numpy as jnp
from jax import lax
from jax.experimental import pallas as pl
from jax.experimental.pallas import tpu as pltpu
from jax.experimental.pallas import tpu_sc as plsc

N0, N1, N2 = 10000, 160000, 50000
BATCH = 128
D = 128
C = 16

N0P = 10240
N1P = 163072
N2P = 51200
E0P = 327680
E1P = 311296
B1P = 327680
B2P = 163840

CHUNK0 = 5120
CHUNK1 = 12544
CAPS = 5264
FT = 4096
GS = 128
GSP = 256
SCAN = 1024

_mesh = plsc.VectorSubcoreMesh(core_axis_name="c", subcore_axis_name="s")


def _zeros16():
    return jnp.zeros((16,), jnp.float32)


def _scatter_kernel_body(nlists, chunk, nchunks, ib, nb, dst_rows,
                         *refs):
    init_hbm = refs[0]
    lists = []
    for i in range(nlists):
        lists.append((refs[1 + 3 * i], refs[2 + 3 * i], refs[3 + 3 * i]))
    y_hbm = refs[1 + 3 * nlists]
    (acc_sp, sscan, dscan, sflat, dflat, sidx, didx, rows,
     sem) = refs[2 + 3 * nlists:]

    c = lax.axis_index("c")
    s = lax.axis_index("s")
    ptr = chunk // 16
    lane = lax.iota(jnp.int32, 16)
    trips = (nchunks - c + 1) // 2

    def chunk_body(j, _):
        lo = (2 * j + c) * chunk
        for bblk in range(nb):
            r0 = s * ptr + bblk * ib
            pltpu.sync_copy(init_hbm.at[pl.ds(lo + r0, ib), :],
                            rows.at[pl.ds(0, ib), :])
            pltpu.sync_copy(rows.at[pl.ds(0, ib), :],
                            acc_sp.at[pl.ds(r0, ib), :])
        plsc.subcore_barrier()

        for (tab, srch, dsth) in lists:
            slice_len = srch.shape[0] // 16
            my0 = s * slice_len

            def gs_loop(nch, tab=tab):
                def gs_body(k, _):
                    kb = k * GS
                    for t in range(GS // 16):
                        sidx[pl.ds(t * 16, 16)] = sflat[pl.ds(kb + t * 16,
                                                              16)]
                        didx[pl.ds(t * 16, 16)] = dflat[pl.ds(kb + t * 16,
                                                              16)]
                    pltpu.async_copy(tab.at[sidx], rows, sem).wait()
                    pltpu.sync_copy(rows, acc_sp.at[didx], add=True)
                    return 0

                lax.fori_loop(0, nch, gs_body, 0)

            def flush(cnt, tab=tab):
                nfull = cnt // GS
                gs_loop(nfull, tab)
                base = nfull * GS
                for t in range(GS // 16):
                    sflat[pl.ds(t * 16, 16)] = sflat[pl.ds(base + t * 16,
                                                           16)]
                    dflat[pl.ds(t * 16, 16)] = dflat[pl.ds(base + t * 16,
                                                           16)]
                return cnt - base

            def vec_body(vi, cnt, lo=lo):
                d = dscan[pl.ds(vi * 16, 16)]
                sv = sscan[pl.ds(vi * 16, 16)]
                m = (d >= lo) & (d < lo + chunk)
                mi = m.astype(jnp.int32)
                incl = plsc.cumsum(mi)
                pos = cnt + incl - mi
                plsc.store_scatter(dflat, [pos], d - lo, mask=m)
                plsc.store_scatter(sflat, [pos], sv, mask=m)
                return cnt + jnp.sum(mi)

            def blk_body(bi, cnt, srch=srch, dsth=dsth, my0=my0, tab=tab):
                base = my0 + bi * SCAN
                pltpu.sync_copy(srch.at[pl.ds(base, SCAN)], sscan)
                pltpu.sync_copy(dsth.at[pl.ds(base, SCAN)], dscan)
                cnt = lax.fori_loop(0, SCAN // 16, vec_body, cnt)
                return lax.cond(cnt >= FT, lambda x: flush(x, tab),
                                lambda x: x, cnt)

            cnt = lax.fori_loop(0, slice_len // SCAN, blk_body, 0)

            dump_vec = chunk + (lane & 7)
            j0 = (cnt // 16) * 16
            rem = cnt - j0
            keep = lane < rem
            dv = dflat[pl.ds(j0, 16)]
            sv = sflat[pl.ds(j0, 16)]
            dflat[pl.ds(j0, 16)] = jnp.where(keep, dv, dump_vec)
            sflat[pl.ds(j0, 16)] = jnp.where(keep, sv, 0)
            pad_end = ((cnt + GS - 1) // GS) * GS
            ntail = jnp.maximum((pad_end - j0 - 16) // 16, 0)

            def pad_body(p, _):
                off = j0 + 16 + p * 16
                dflat[pl.ds(off, 16)] = dump_vec
                sflat[pl.ds(off, 16)] = jnp.zeros((16,), jnp.int32)
                return 0

            lax.fori_loop(0, ntail, pad_body, 0)
            gs_loop(pad_end // GS, tab)

        plsc.subcore_barrier()
        for bblk in range(nb):
            r0 = s * ptr + bblk * ib
            pltpu.sync_copy(acc_sp.at[pl.ds(r0, ib), :],
                            rows.at[pl.ds(0, ib), :])
            pltpu.sync_copy(rows.at[pl.ds(0, ib), :],
                            y_hbm.at[pl.ds(lo + r0, ib), :])
        return 0

    lax.fori_loop(0, trips, chunk_body, 0)


def _make_scatter_kernel(nlists, chunk, nchunks, ib, nb, dst_rows):
    body = functools.partial(_scatter_kernel_body, nlists, chunk, nchunks,
                             ib, nb, dst_rows)
    return pl.kernel(
        body,
        out_type=jax.ShapeDtypeStruct((dst_rows, D), jnp.float32),
        mesh=_mesh,
        compiler_params=pltpu.CompilerParams(needs_layout_passes=False),
        scratch_types=[
            pltpu.VMEM_SHARED((chunk + 8, D), jnp.float32),
            pltpu.VMEM((SCAN,), jnp.int32),
            pltpu.VMEM((SCAN,), jnp.int32),
            pltpu.VMEM((CAPS,), jnp.int32),
            pltpu.VMEM((CAPS,), jnp.int32),
            pltpu.VMEM((GS,), jnp.int32),
            pltpu.VMEM((GS,), jnp.int32),
            pltpu.VMEM((GS, D), jnp.float32),
            pltpu.SemaphoreType.DMA,
        ],
    )


def _pool_kernel_body(y0, y1, x1, x2p,
                      s0, d0, s1, d1, bs1, bd1, bs2, bd2,
                      b0p, b1p, b2p, out_hbm,
                      acc, sidx, didx, bkt, rows, redout, grid_sp,
                      sem, sem2):
    c = lax.axis_index("c")
    s = lax.axis_index("s")
    wid = c * 16 + s

    def z_body(r, _):
        for jj in range(8):
            acc[r, pl.ds(jj * 16, 16)] = _zeros16()
        return 0

    lax.fori_loop(0, BATCH + 8, z_body, 0)

    def accum(nrows, rows_ref, bkt_ref):
        def row_body(i, _):
            g = bkt_ref[pl.ds(i, 16)][0]
            for jj in range(8):
                col = jj * 16
                acc[g, pl.ds(col, 16)] = (acc[g, pl.ds(col, 16)]
                                          + rows_ref[i, pl.ds(col, 16)])
            return 0

        lax.fori_loop(0, nrows, row_body, 0)

    bucket_lists = [
        (y0, s0, d0, b0p),
        (y1, s1, d1, b1p),
        (y0, bs1, bd1, b1p),
        (x1, bs2, bd2, b2p),
        (y1, bs2, bd2, b2p),
    ]
    for (tab, srch, dsth, bat) in bucket_lists:
        n = srch.shape[0] // 32
        base0 = wid * n

        def blk(k, _, tab=tab, srch=srch, dsth=dsth, bat=bat, base0=base0):
            bpos = base0 + k * GSP
            pltpu.sync_copy(srch.at[pl.ds(bpos, GSP)], sidx)
            pltpu.sync_copy(dsth.at[pl.ds(bpos, GSP)], didx)
            pltpu.async_copy(bat.at[didx], bkt.at[pl.ds(0, GSP)], sem).wait()
            pltpu.async_copy(tab.at[sidx], rows, sem2).wait()
            accum(GSP, rows, bkt)
            return 0

        lax.fori_loop(0, n // GSP, blk, 0)

    linear_lists = [
        (y0, b0p, N0P, 64),
        (y1, b1p, N1P, 56),
        (x2p, b2p, N2P, 200),
    ]
    for (tab, bat, tot, blkrows) in linear_lists:
        n = tot // 32
        base0 = wid * n

        def lblk(k, _, tab=tab, bat=bat, base0=base0, blkrows=blkrows):
            r0 = base0 + k * blkrows
            pltpu.sync_copy(tab.at[pl.ds(r0, blkrows), :],
                            rows.at[pl.ds(0, blkrows), :])
            pltpu.sync_copy(bat.at[pl.ds(r0, blkrows)],
                            bkt.at[pl.ds(0, blkrows)])
            accum(blkrows, rows, bkt)
            return 0

        lax.fori_loop(0, n // blkrows, lblk, 0)

    pltpu.sync_copy(acc, grid_sp.at[s])
    plsc.subcore_barrier()
    for si in range(16):
        pltpu.sync_copy(grid_sp.at[si, pl.ds(8 * s, 8), :],
                        rows.at[pl.ds(si * 8, 8), :])

    def rz_body(r, _):
        for jj in range(8):
            redout[r, pl.ds(jj * 16, 16)] = _zeros16()
        return 0

    lax.fori_loop(0, 8, rz_body, 0)

    def red_body(si, _):
        for r in range(8):
            for jj in range(8):
                col = jj * 16
                redout[r, pl.ds(col, 16)] = (
                    redout[r, pl.ds(col, 16)]
                    + rows[si * 8 + r, pl.ds(col, 16)])
        return 0

    lax.fori_loop(0, 16, red_body, 0)
    pltpu.sync_copy(redout, out_hbm.at[c, pl.ds(8 * s, 8), :])


_pool_kernel = pl.kernel(
    _pool_kernel_body,
    out_type=jax.ShapeDtypeStruct((2, BATCH, D), jnp.float32),
    mesh=_mesh,
    compiler_params=pltpu.CompilerParams(needs_layout_passes=False),
    scratch_types=[
        pltpu.VMEM((BATCH + 8, D), jnp.float32),
        pltpu.VMEM((GSP,), jnp.int32),
        pltpu.VMEM((GSP,), jnp.int32),
        pltpu.VMEM((GSP + 16,), jnp.int32),
        pltpu.VMEM((GSP, D), jnp.float32),
        pltpu.VMEM((8, D), jnp.float32),
        pltpu.VMEM_SHARED((16, BATCH + 8, D), jnp.float32),
        pltpu.SemaphoreType.DMA,
        pltpu.SemaphoreType.DMA,
    ],
)


def _head_body(part_ref, w_ref, b_ref, out_ref):
    pooled = part_ref[0] + part_ref[1]
    logits = jnp.dot(pooled, w_ref[...],
                     preferred_element_type=jnp.float32) + b_ref[...]
    mx = jnp.max(logits, axis=-1, keepdims=True)
    sh = logits - mx
    lse = jnp.log(jnp.sum(jnp.exp(sh), axis=-1, keepdims=True))
    out_ref[...] = sh - lse


def _pad1(a, total, fill):
    pad = total - a.shape[0]
    return jnp.concatenate([a.astype(jnp.int32),
                            fill.astype(jnp.int32)[:pad]])


def kernel(x0, x1, x2, up_index0, up_index1, b1_src, b1_dst, b2_src, b2_dst,
           batch0, batch1, batch2, W, b):
    f32 = jnp.float32
    x0p = jnp.concatenate([x0, jnp.zeros((N0P - N0, D), f32)])
    x2p = jnp.concatenate([x2, jnp.zeros((N2P - N2, D), f32)])

    def spread(n_pad, base, mod):
        return base + (jnp.arange(n_pad, dtype=jnp.int32) % mod)

    s0 = _pad1(up_index0[0], E0P, spread(E0P - 320000, 0, N0))
    d0 = _pad1(up_index0[1], E0P, spread(E0P - 320000, N0, N0P - N0))
    s1 = _pad1(up_index1[0], E1P, spread(E1P - 300000, 0, N1))
    d1 = _pad1(up_index1[1], E1P, spread(E1P - 300000, N1, N1P - N1))
    bs1 = _pad1(b1_src, B1P, spread(B1P - 320000, 0, N0))
    bd1 = _pad1(b1_dst, B1P, spread(B1P - 320000, N1, N1P - N1))
    bs2 = _pad1(b2_src, B2P, spread(B2P - 150000, 0, N1))
    bd2 = _pad1(b2_dst, B2P, spread(B2P - 150000, N2, N2P - N2))

    b0p = _pad1(batch0, N0P, jnp.full((N0P - N0,), BATCH, jnp.int32))
    b1p = _pad1(batch1, N1P, jnp.full((N1P - N1,), BATCH, jnp.int32))
    b2p = _pad1(batch2, N2P, jnp.full((N2P - N2,), BATCH, jnp.int32))

    k1 = _make_scatter_kernel(1, CHUNK0, 2, 64, 5, N0P)
    y0 = k1(x0p, x0p, s0, d0)

    x1p = jnp.concatenate([x1, jnp.zeros((N1P - N1, D), f32)])
    k2 = _make_scatter_kernel(2, CHUNK1, 13, 112, 7, N1P)
    y1 = k2(x1p, x1p, s1, d1, x0, bs1, bd1)

    part = _pool_kernel(y0, y1, x1, x2p, s0, d0, s1, d1, bs1, bd1, bs2, bd2,
                        b0p, b1p, b2p)

    return pl.pallas_call(
        _head_body,
        out_shape=jax.ShapeDtypeStruct((BATCH, C), f32),
    )(part, W, b.reshape(1, C))


# speedup vs baseline: 1.7052x; 1.2194x over previous
import functools

import jax
import jax.numpy as jnp
from jax import lax
from jax.experimental import pallas as pl
from jax.experimental.pallas import tpu as pltpu
from jax.experimental.pallas import tpu_sc as plsc

N0, N1, N2 = 10000, 160000, 50000
BATCH = 128
D = 128
C = 16

N0P = 10240
N1P = 172032
N2P = 51200
E0P = 327680
E1P = 311296
B1P = 327680
B2P = 163840

CHUNK0 = 5120
CHUNK1 = 12288
CAPS = 5264
FT = 4096
GS = 128
GSP = 256
LB = 64
SCAN = 1024

OFF_Y1 = N0P
OFF_X1 = N0P + N1P
OFF_X2 = N0P + N1P + N1
NT = N0P + N1P + N1 + N2P
OFF_B1 = N0P
OFF_B2 = N0P + N1P
NBT = N0P + N1P + N2P
EALL = E0P + E1P + B1P + 2 * B2P

_mesh = plsc.VectorSubcoreMesh(core_axis_name="c", subcore_axis_name="s")


def _zeros16():
    return jnp.zeros((16,), jnp.float32)


def _scatter_kernel_body(nlists, chunk, nchunks, dst_rows, *refs):
    init_hbm = refs[0]
    lists = []
    for i in range(nlists):
        lists.append((refs[1 + 3 * i], refs[2 + 3 * i], refs[3 + 3 * i]))
    y_hbm = refs[1 + 3 * nlists]
    (acc_sp, sscan, dscan, sflat, dflat, sidx, didx, rows,
     sem, semA, semB) = refs[2 + 3 * nlists:]

    c = lax.axis_index("c")
    s = lax.axis_index("s")
    ptr = chunk // 16
    lane = lax.iota(jnp.int32, 16)
    trips = (nchunks - c + 1) // 2

    def chunk_body(j, _):
        lo = (2 * j + c) * chunk
        r0 = s * ptr
        pltpu.sync_copy(init_hbm.at[pl.ds(lo + r0, ptr), :],
                        acc_sp.at[pl.ds(r0, ptr), :])
        plsc.subcore_barrier()

        for (tab, srch, dsth) in lists:
            slice_len = srch.shape[0] // 16
            my0 = s * slice_len
            nblk = slice_len // SCAN

            def gs_loop(nch, tab=tab):
                def gs_body(k, _):
                    kb = k * GS
                    for t in range(GS // 16):
                        sidx[pl.ds(t * 16, 16)] = sflat[pl.ds(kb + t * 16,
                                                              16)]
                        didx[pl.ds(t * 16, 16)] = dflat[pl.ds(kb + t * 16,
                                                              16)]
                    pltpu.async_copy(tab.at[sidx], rows, sem).wait()
                    pltpu.sync_copy(rows, acc_sp.at[didx], add=True)
                    return 0

                lax.fori_loop(0, nch, gs_body, 0)

            def flush(cnt, tab=tab):
                nfull = cnt // GS
                gs_loop(nfull, tab)
                base = nfull * GS
                for t in range(GS // 16):
                    sflat[pl.ds(t * 16, 16)] = sflat[pl.ds(base + t * 16,
                                                           16)]
                    dflat[pl.ds(t * 16, 16)] = dflat[pl.ds(base + t * 16,
                                                           16)]
                return cnt - base

            def vec_body(vi, cnt, lo=lo):
                d1 = dscan[pl.ds(vi * 32, 16)]
                d2 = dscan[pl.ds(vi * 32 + 16, 16)]
                s1 = sscan[pl.ds(vi * 32, 16)]
                s2 = sscan[pl.ds(vi * 32 + 16, 16)]
                m1 = (d1 >= lo) & (d1 < lo + chunk)
                m2 = (d2 >= lo) & (d2 < lo + chunk)
                mi1 = m1.astype(jnp.int32)
                mi2 = m2.astype(jnp.int32)
                i1 = plsc.cumsum(mi1)
                i2 = plsc.cumsum(mi2)
                t1 = i1[15]
                pos1 = cnt + i1 - mi1
                pos2 = cnt + t1 + i2 - mi2
                plsc.store_scatter(dflat, [pos1], d1 - lo, mask=m1)
                plsc.store_scatter(sflat, [pos1], s1, mask=m1)
                plsc.store_scatter(dflat, [pos2], d2 - lo, mask=m2)
                plsc.store_scatter(sflat, [pos2], s2, mask=m2)
                return cnt + t1 + i2[15]

            def scan_one(cnt, half, tab=tab):
                off = half * (SCAN // 32)
                cnt = lax.fori_loop(0, SCAN // 32,
                                    lambda vi, cn: vec_body(vi + off, cn),
                                    cnt)
                return lax.cond(cnt >= FT, lambda x: flush(x, tab),
                                lambda x: x, cnt)

            def pair_body(q, cnt, srch=srch, dsth=dsth, my0=my0, tab=tab):
                b0 = my0 + q * 2 * SCAN
                cs0 = pltpu.async_copy(srch.at[pl.ds(b0, SCAN)],
                                       sscan.at[pl.ds(0, SCAN)], semA)
                cd0 = pltpu.async_copy(dsth.at[pl.ds(b0, SCAN)],
                                       dscan.at[pl.ds(0, SCAN)], semA)
                cs1 = pltpu.async_copy(srch.at[pl.ds(b0 + SCAN, SCAN)],
                                       sscan.at[pl.ds(SCAN, SCAN)], semB)
                cd1 = pltpu.async_copy(dsth.at[pl.ds(b0 + SCAN, SCAN)],
                                       dscan.at[pl.ds(SCAN, SCAN)], semB)
                cs0.wait()
                cd0.wait()
                cnt = scan_one(cnt, 0, tab)
                cs1.wait()
                cd1.wait()
                cnt = scan_one(cnt, 1, tab)
                return cnt

            cnt = lax.fori_loop(0, nblk // 2, pair_body, 0)
            if nblk % 2:
                b0 = my0 + (nblk - 1) * SCAN
                pltpu.sync_copy(srch.at[pl.ds(b0, SCAN)],
                                sscan.at[pl.ds(0, SCAN)])
                pltpu.sync_copy(dsth.at[pl.ds(b0, SCAN)],
                                dscan.at[pl.ds(0, SCAN)])
                cnt = scan_one(cnt, 0, tab)

            dump_vec = chunk + (lane & 7)
            j0 = (cnt // 16) * 16
            rem = cnt - j0
            keep = lane < rem
            dv = dflat[pl.ds(j0, 16)]
            sv = sflat[pl.ds(j0, 16)]
            dflat[pl.ds(j0, 16)] = jnp.where(keep, dv, dump_vec)
            sflat[pl.ds(j0, 16)] = jnp.where(keep, sv, 0)
            pad_end = ((cnt + GS - 1) // GS) * GS
            ntail = jnp.maximum((pad_end - j0 - 16) // 16, 0)

            def pad_body(p, _):
                off = j0 + 16 + p * 16
                dflat[pl.ds(off, 16)] = dump_vec
                sflat[pl.ds(off, 16)] = jnp.zeros((16,), jnp.int32)
                return 0

            lax.fori_loop(0, ntail, pad_body, 0)
            gs_loop(pad_end // GS, tab)

        plsc.subcore_barrier()
        pltpu.sync_copy(acc_sp.at[pl.ds(r0, ptr), :],
                        y_hbm.at[pl.ds(lo + r0, ptr), :])
        return 0

    lax.fori_loop(0, trips, chunk_body, 0)


def _make_scatter_kernel(nlists, chunk, nchunks, dst_rows):
    body = functools.partial(_scatter_kernel_body, nlists, chunk, nchunks,
                             dst_rows)
    return pl.kernel(
        body,
        out_type=jax.ShapeDtypeStruct((dst_rows, D), jnp.float32),
        mesh=_mesh,
        compiler_params=pltpu.CompilerParams(needs_layout_passes=False),
        scratch_types=[
            pltpu.VMEM_SHARED((chunk + 8, D), jnp.float32),
            pltpu.VMEM((2 * SCAN,), jnp.int32),
            pltpu.VMEM((2 * SCAN,), jnp.int32),
            pltpu.VMEM((CAPS,), jnp.int32),
            pltpu.VMEM((CAPS,), jnp.int32),
            pltpu.VMEM((GS,), jnp.int32),
            pltpu.VMEM((GS,), jnp.int32),
            pltpu.VMEM((GS, D), jnp.float32),
            pltpu.SemaphoreType.DMA,
            pltpu.SemaphoreType.DMA,
            pltpu.SemaphoreType.DMA,
        ],
    )


def _pool_kernel_body(tabl, esall, edall, ball, out_hbm,
                      acc, sidx_a, sidx_b, didx_a, didx_b, bkt_a, bkt_b,
                      rows_a, rows_b, redout, grid_sp,
                      semA, semB, semC, semD):
    c = lax.axis_index("c")
    s = lax.axis_index("s")
    wid = c * 16 + s

    def z_body(r, _):
        for jj in range(8):
            acc[r, pl.ds(jj * 16, 16)] = _zeros16()
        return 0

    lax.fori_loop(0, BATCH + 8, z_body, 0)

    def accum(bkt_ref, rows_ref, nrows):
        def grp_body(g, _):
            i0 = g * 16
            bv = bkt_ref[pl.ds(i0, 16)]
            for l in range(16):
                gb = bv[l]
                for jj in range(8):
                    col = jj * 16
                    acc[gb, pl.ds(col, 16)] = (
                        acc[gb, pl.ds(col, 16)]
                        + rows_ref[i0 + l, pl.ds(col, 16)])
            return 0

        lax.fori_loop(0, nrows // 16, grp_body, 0)

    n = EALL // 32
    base0 = wid * n
    nblk = n // GSP

    def bpair(q, _):
        b0 = base0 + q * 2 * GSP
        ds0 = [pltpu.async_copy(esall.at[pl.ds(b0, GSP)], sidx_a, semA),
               pltpu.async_copy(edall.at[pl.ds(b0, GSP)], didx_a, semA)]
        ds1 = [pltpu.async_copy(esall.at[pl.ds(b0 + GSP, GSP)], sidx_b,
                                semB),
               pltpu.async_copy(edall.at[pl.ds(b0 + GSP, GSP)], didx_b,
                                semB)]
        for d_ in ds0:
            d_.wait()
        g0 = [pltpu.async_copy(ball.at[didx_a], bkt_a.at[pl.ds(0, GSP)],
                               semC),
              pltpu.async_copy(tabl.at[sidx_a], rows_a, semC)]
        for d_ in ds1:
            d_.wait()
        g1 = [pltpu.async_copy(ball.at[didx_b], bkt_b.at[pl.ds(0, GSP)],
                               semD),
              pltpu.async_copy(tabl.at[sidx_b], rows_b, semD)]
        for d_ in g0:
            d_.wait()
        accum(bkt_a, rows_a, GSP)
        for d_ in g1:
            d_.wait()
        accum(bkt_b, rows_b, GSP)
        return 0

    lax.fori_loop(0, nblk // 2, bpair, 0)

    for (toff, boff, tot) in ((0, 0, N0P + N1P),
                              (OFF_X2, OFF_B2, N2P)):
        npt = tot // 32
        lb0 = toff + wid * npt
        bb0 = boff + wid * npt
        nlb = npt // LB

        def lpair(q, _, lb0=lb0, bb0=bb0):
            r0 = lb0 + q * 2 * LB
            r1 = bb0 + q * 2 * LB
            g0 = [pltpu.async_copy(tabl.at[pl.ds(r0, LB), :],
                                   rows_a.at[pl.ds(0, LB), :], semC),
                  pltpu.async_copy(ball.at[pl.ds(r1, LB)],
                                   bkt_a.at[pl.ds(0, LB)], semC)]
            g1 = [pltpu.async_copy(tabl.at[pl.ds(r0 + LB, LB), :],
                                   rows_b.at[pl.ds(0, LB), :], semD),
                  pltpu.async_copy(ball.at[pl.ds(r1 + LB, LB)],
                                   bkt_b.at[pl.ds(0, LB)], semD)]
            for d_ in g0:
                d_.wait()
            accum(bkt_a, rows_a, LB)
            for d_ in g1:
                d_.wait()
            accum(bkt_b, rows_b, LB)
            return 0

        lax.fori_loop(0, nlb // 2, lpair, 0)
        if nlb % 2:
            r0 = lb0 + (nlb - 1) * LB
            r1 = bb0 + (nlb - 1) * LB
            pltpu.sync_copy(tabl.at[pl.ds(r0, LB), :],
                            rows_a.at[pl.ds(0, LB), :])
            pltpu.sync_copy(ball.at[pl.ds(r1, LB)],
                            bkt_a.at[pl.ds(0, LB)])
            accum(bkt_a, rows_a, LB)

    pltpu.sync_copy(acc, grid_sp.at[s])
    plsc.subcore_barrier()
    for si in range(16):
        pltpu.sync_copy(grid_sp.at[si, pl.ds(8 * s, 8), :],
                        rows_a.at[pl.ds(si * 8, 8), :])

    def rz_body(r, _):
        for jj in range(8):
            redout[r, pl.ds(jj * 16, 16)] = _zeros16()
        return 0

    lax.fori_loop(0, 8, rz_body, 0)

    def red_body(si, _):
        for r in range(8):
            for jj in range(8):
                col = jj * 16
                redout[r, pl.ds(col, 16)] = (
                    redout[r, pl.ds(col, 16)]
                    + rows_a[si * 8 + r, pl.ds(col, 16)])
        return 0

    lax.fori_loop(0, 16, red_body, 0)
    pltpu.sync_copy(redout, out_hbm.at[c, pl.ds(8 * s, 8), :])


_pool_kernel = pl.kernel(
    _pool_kernel_body,
    out_type=jax.ShapeDtypeStruct((2, BATCH, D), jnp.float32),
    mesh=_mesh,
    compiler_params=pltpu.CompilerParams(needs_layout_passes=False),
    scratch_types=[
        pltpu.VMEM((BATCH + 8, D), jnp.float32),
        pltpu.VMEM((GSP,), jnp.int32),
        pltpu.VMEM((GSP,), jnp.int32),
        pltpu.VMEM((GSP,), jnp.int32),
        pltpu.VMEM((GSP,), jnp.int32),
        pltpu.VMEM((GSP + 16,), jnp.int32),
        pltpu.VMEM((GSP + 16,), jnp.int32),
        pltpu.VMEM((GSP, D), jnp.float32),
        pltpu.VMEM((GSP, D), jnp.float32),
        pltpu.VMEM((8, D), jnp.float32),
        pltpu.VMEM_SHARED((16, BATCH + 8, D), jnp.float32),
        pltpu.SemaphoreType.DMA,
        pltpu.SemaphoreType.DMA,
        pltpu.SemaphoreType.DMA,
        pltpu.SemaphoreType.DMA,
    ],
)


def _head_body(part_ref, w_ref, b_ref, out_ref):
    pooled = part_ref[0] + part_ref[1]
    logits = jnp.dot(pooled, w_ref[...],
                     preferred_element_type=jnp.float32) + b_ref[...]
    mx = jnp.max(logits, axis=-1, keepdims=True)
    sh = logits - mx
    lse = jnp.log(jnp.sum(jnp.exp(sh), axis=-1, keepdims=True))
    out_ref[...] = sh - lse


def _pad1(a, total, fill):
    pad = total - a.shape[0]
    return jnp.concatenate([a.astype(jnp.int32),
                            fill.astype(jnp.int32)[:pad]])


def kernel(x0, x1, x2, up_index0, up_index1, b1_src, b1_dst, b2_src, b2_dst,
           batch0, batch1, batch2, W, b):
    f32 = jnp.float32
    x0p = jnp.concatenate([x0, jnp.zeros((N0P - N0, D), f32)])
    x1p = jnp.concatenate([x1, jnp.zeros((N1P - N1, D), f32)])
    x2p = jnp.concatenate([x2, jnp.zeros((N2P - N2, D), f32)])

    def spread(n_pad, base, mod):
        return base + (jnp.arange(n_pad, dtype=jnp.int32) % mod)

    s0 = _pad1(up_index0[0], E0P, spread(E0P - 320000, 0, N0))
    d0 = _pad1(up_index0[1], E0P, spread(E0P - 320000, N0, N0P - N0))
    s1 = _pad1(up_index1[0], E1P, spread(E1P - 300000, 0, N1))
    d1 = _pad1(up_index1[1], E1P, spread(E1P - 300000, N1, N1P - N1))
    bs1 = _pad1(b1_src, B1P, spread(B1P - 320000, 0, N0))
    bd1 = _pad1(b1_dst, B1P, spread(B1P - 320000, N1, N1P - N1))
    bs2 = _pad1(b2_src, B2P, spread(B2P - 150000, 0, N1))
    bd2 = _pad1(b2_dst, B2P, spread(B2P - 150000, N2, N2P - N2))

    b0p = _pad1(batch0, N0P, jnp.full((N0P - N0,), BATCH, jnp.int32))
    b1p = _pad1(batch1, N1P, jnp.full((N1P - N1,), BATCH, jnp.int32))
    b2p = _pad1(batch2, N2P, jnp.full((N2P - N2,), BATCH, jnp.int32))

    k1 = _make_scatter_kernel(1, CHUNK0, 2, N0P)
    y0 = k1(x0p, x0p, s0, d0)

    k2 = _make_scatter_kernel(2, CHUNK1, 14, N1P)
    y1 = k2(x1p, x1p, s1, d1, x0, bs1, bd1)

    tabl = jnp.concatenate([y0, y1, x1, x2p])
    ball = jnp.concatenate([b0p, b1p, b2p])
    esall = jnp.concatenate([s0, s1 + OFF_Y1, bs1, bs2 + OFF_Y1,
                             bs2 + OFF_X1])
    edall = jnp.concatenate([d0, d1 + OFF_B1, bd1 + OFF_B1, bd2 + OFF_B2,
                             bd2 + OFF_B2])

    part = _pool_kernel(tabl, esall, edall, ball)

    return pl.pallas_call(
        _head_body,
        out_shape=jax.ShapeDtypeStruct((BATCH, C), f32),
    )(part, W, b.reshape(1, C))


# speedup vs baseline: 1.9044x; 1.1168x over previous
import functools

import jax
import jax.numpy as jnp
from jax import lax
from jax.experimental import pallas as pl
from jax.experimental.pallas import tpu as pltpu
from jax.experimental.pallas import tpu_sc as plsc

N0, N1, N2 = 10000, 160000, 50000
BATCH = 128
D = 128
C = 16

N0P = 10240
N1P = 172032
N2P = 51200
E0P = 327680
E1P = 311296
B1P = 327680
B2P = 163840

CHUNK0 = 5120
CHUNK1 = 12288
CAPS = 5264
FT = 4096
GS = 128
GSP = 256
LB = 64
SCAN = 1024

OFF_Y1 = N0P
OFF_X1 = N0P + N1P
OFF_X2 = N0P + N1P + N1
NT = N0P + N1P + N1 + N2P
OFF_B1 = N0P
OFF_B2 = N0P + N1P
NBT = N0P + N1P + N2P
EALL = E0P + E1P + B1P + 2 * B2P

_mesh = plsc.VectorSubcoreMesh(core_axis_name="c", subcore_axis_name="s")


def _zeros16():
    return jnp.zeros((16,), jnp.float32)


def _scatter_kernel_body(nlists, chunk, nchunks, dst_rows, *refs):
    init_hbm = refs[0]
    lists = []
    for i in range(nlists):
        lists.append((refs[1 + 3 * i], refs[2 + 3 * i], refs[3 + 3 * i]))
    y_hbm = refs[1 + 3 * nlists]
    (acc_sp, sscan, dscan, sflat, dflat, sidx, didx, rows,
     sem, semA, semB) = refs[2 + 3 * nlists:]

    c = lax.axis_index("c")
    s = lax.axis_index("s")
    ptr = chunk // 16
    lane = lax.iota(jnp.int32, 16)
    trips = (nchunks - c + 1) // 2

    def chunk_body(j, _):
        lo = (2 * j + c) * chunk
        r0 = s * ptr
        pltpu.sync_copy(init_hbm.at[pl.ds(lo + r0, ptr), :],
                        acc_sp.at[pl.ds(r0, ptr), :])
        plsc.subcore_barrier()

        for (tab, srch, dsth) in lists:
            slice_len = srch.shape[0] // 16
            my0 = s * slice_len
            nblk = slice_len // SCAN

            def gs_loop(nch, tab=tab):
                def gs_body(k, _):
                    kb = k * GS
                    for t in range(GS // 16):
                        sidx[pl.ds(t * 16, 16)] = sflat[pl.ds(kb + t * 16,
                                                              16)]
                        didx[pl.ds(t * 16, 16)] = dflat[pl.ds(kb + t * 16,
                                                              16)]
                    pltpu.async_copy(tab.at[sidx], rows, sem).wait()
                    pltpu.sync_copy(rows, acc_sp.at[didx], add=True)
                    return 0

                lax.fori_loop(0, nch, gs_body, 0)

            def flush(cnt, tab=tab):
                nfull = cnt // GS
                gs_loop(nfull, tab)
                base = nfull * GS
                for t in range(GS // 16):
                    sflat[pl.ds(t * 16, 16)] = sflat[pl.ds(base + t * 16,
                                                           16)]
                    dflat[pl.ds(t * 16, 16)] = dflat[pl.ds(base + t * 16,
                                                           16)]
                return cnt - base

            def vec_body(vi, cnt, lo=lo):
                d1 = dscan[pl.ds(vi * 32, 16)]
                d2 = dscan[pl.ds(vi * 32 + 16, 16)]
                s1 = sscan[pl.ds(vi * 32, 16)]
                s2 = sscan[pl.ds(vi * 32 + 16, 16)]
                m1 = (d1 >= lo) & (d1 < lo + chunk)
                m2 = (d2 >= lo) & (d2 < lo + chunk)
                mi1 = m1.astype(jnp.int32)
                mi2 = m2.astype(jnp.int32)
                i1 = plsc.cumsum(mi1)
                i2 = plsc.cumsum(mi2)
                t1 = i1[15]
                pos1 = cnt + i1 - mi1
                pos2 = cnt + t1 + i2 - mi2
                plsc.store_scatter(dflat, [pos1], d1 - lo, mask=m1)
                plsc.store_scatter(sflat, [pos1], s1, mask=m1)
                plsc.store_scatter(dflat, [pos2], d2 - lo, mask=m2)
                plsc.store_scatter(sflat, [pos2], s2, mask=m2)
                return cnt + t1 + i2[15]

            def scan_one(cnt, half, tab=tab):
                off = half * (SCAN // 32)
                cnt = lax.fori_loop(0, SCAN // 32,
                                    lambda vi, cn: vec_body(vi + off, cn),
                                    cnt)
                return lax.cond(cnt >= FT, lambda x: flush(x, tab),
                                lambda x: x, cnt)

            def pair_body(q, cnt, srch=srch, dsth=dsth, my0=my0, tab=tab):
                b0 = my0 + q * 2 * SCAN
                cs0 = pltpu.async_copy(srch.at[pl.ds(b0, SCAN)],
                                       sscan.at[pl.ds(0, SCAN)], semA)
                cd0 = pltpu.async_copy(dsth.at[pl.ds(b0, SCAN)],
                                       dscan.at[pl.ds(0, SCAN)], semA)
                cs1 = pltpu.async_copy(srch.at[pl.ds(b0 + SCAN, SCAN)],
                                       sscan.at[pl.ds(SCAN, SCAN)], semB)
                cd1 = pltpu.async_copy(dsth.at[pl.ds(b0 + SCAN, SCAN)],
                                       dscan.at[pl.ds(SCAN, SCAN)], semB)
                cs0.wait()
                cd0.wait()
                cnt = scan_one(cnt, 0, tab)
                cs1.wait()
                cd1.wait()
                cnt = scan_one(cnt, 1, tab)
                return cnt

            cnt = lax.fori_loop(0, nblk // 2, pair_body, 0)
            if nblk % 2:
                b0 = my0 + (nblk - 1) * SCAN
                pltpu.sync_copy(srch.at[pl.ds(b0, SCAN)],
                                sscan.at[pl.ds(0, SCAN)])
                pltpu.sync_copy(dsth.at[pl.ds(b0, SCAN)],
                                dscan.at[pl.ds(0, SCAN)])
                cnt = scan_one(cnt, 0, tab)

            dump_vec = chunk + (lane & 7)
            j0 = (cnt // 16) * 16
            rem = cnt - j0
            keep = lane < rem
            dv = dflat[pl.ds(j0, 16)]
            sv = sflat[pl.ds(j0, 16)]
            dflat[pl.ds(j0, 16)] = jnp.where(keep, dv, dump_vec)
            sflat[pl.ds(j0, 16)] = jnp.where(keep, sv, 0)
            pad_end = ((cnt + GS - 1) // GS) * GS
            ntail = jnp.maximum((pad_end - j0 - 16) // 16, 0)

            def pad_body(p, _):
                off = j0 + 16 + p * 16
                dflat[pl.ds(off, 16)] = dump_vec
                sflat[pl.ds(off, 16)] = jnp.zeros((16,), jnp.int32)
                return 0

            lax.fori_loop(0, ntail, pad_body, 0)
            gs_loop(pad_end // GS, tab)

        plsc.subcore_barrier()
        pltpu.sync_copy(acc_sp.at[pl.ds(r0, ptr), :],
                        y_hbm.at[pl.ds(lo + r0, ptr), :])
        return 0

    lax.fori_loop(0, trips, chunk_body, 0)


def _make_scatter_kernel(nlists, chunk, nchunks, dst_rows):
    body = functools.partial(_scatter_kernel_body, nlists, chunk, nchunks,
                             dst_rows)
    return pl.kernel(
        body,
        out_type=jax.ShapeDtypeStruct((dst_rows, D), jnp.float32),
        mesh=_mesh,
        compiler_params=pltpu.CompilerParams(needs_layout_passes=False),
        scratch_types=[
            pltpu.VMEM_SHARED((chunk + 8, D), jnp.float32),
            pltpu.VMEM((2 * SCAN,), jnp.int32),
            pltpu.VMEM((2 * SCAN,), jnp.int32),
            pltpu.VMEM((CAPS,), jnp.int32),
            pltpu.VMEM((CAPS,), jnp.int32),
            pltpu.VMEM((GS,), jnp.int32),
            pltpu.VMEM((GS,), jnp.int32),
            pltpu.VMEM((GS, D), jnp.float32),
            pltpu.SemaphoreType.DMA,
            pltpu.SemaphoreType.DMA,
            pltpu.SemaphoreType.DMA,
        ],
    )


def _pool_kernel_body(tabl, esall, edall, ball, out_hbm,
                      acc, sidx_a, sidx_b, didx_a, didx_b, bkt_a, bkt_b,
                      rows_a, rows_b, redout, grid_sp,
                      semA, semB, semC, semD):
    c = lax.axis_index("c")
    s = lax.axis_index("s")
    wid = c * 16 + s

    def z_body(r, _):
        for jj in range(8):
            acc[r, pl.ds(jj * 16, 16)] = _zeros16()
        return 0

    lax.fori_loop(0, BATCH + 8, z_body, 0)

    def accum(bkt_ref, rows_ref, nrows):
        def grp_body(g, _):
            i0 = g * 16
            bv = bkt_ref[pl.ds(i0, 16)]
            for l in range(16):
                gb = bv[l]
                for jj in range(8):
                    col = jj * 16
                    plsc.addupdate(acc.at[gb, pl.ds(col, 16)],
                                   rows_ref[i0 + l, pl.ds(col, 16)])
            return 0

        lax.fori_loop(0, nrows // 16, grp_body, 0)

    n = EALL // 32
    base0 = wid * n
    nblk = n // GSP

    def bpair(q, _):
        b0 = base0 + q * 2 * GSP
        ds0 = [pltpu.async_copy(esall.at[pl.ds(b0, GSP)], sidx_a, semA),
               pltpu.async_copy(edall.at[pl.ds(b0, GSP)], didx_a, semA)]
        ds1 = [pltpu.async_copy(esall.at[pl.ds(b0 + GSP, GSP)], sidx_b,
                                semB),
               pltpu.async_copy(edall.at[pl.ds(b0 + GSP, GSP)], didx_b,
                                semB)]
        for d_ in ds0:
            d_.wait()
        g0 = [pltpu.async_copy(ball.at[didx_a], bkt_a.at[pl.ds(0, GSP)],
                               semC),
              pltpu.async_copy(tabl.at[sidx_a], rows_a, semC)]
        for d_ in ds1:
            d_.wait()
        g1 = [pltpu.async_copy(ball.at[didx_b], bkt_b.at[pl.ds(0, GSP)],
                               semD),
              pltpu.async_copy(tabl.at[sidx_b], rows_b, semD)]
        for d_ in g0:
            d_.wait()
        accum(bkt_a, rows_a, GSP)
        for d_ in g1:
            d_.wait()
        accum(bkt_b, rows_b, GSP)
        return 0

    lax.fori_loop(0, nblk // 2, bpair, 0)

    for (toff, boff, tot) in ((0, 0, N0P + N1P),
                              (OFF_X2, OFF_B2, N2P)):
        npt = tot // 32
        lb0 = toff + wid * npt
        bb0 = boff + wid * npt
        nlb = npt // LB

        def lpair(q, _, lb0=lb0, bb0=bb0):
            r0 = lb0 + q * 2 * LB
            r1 = bb0 + q * 2 * LB
            g0 = [pltpu.async_copy(tabl.at[pl.ds(r0, LB), :],
                                   rows_a.at[pl.ds(0, LB), :], semC),
                  pltpu.async_copy(ball.at[pl.ds(r1, LB)],
                                   bkt_a.at[pl.ds(0, LB)], semC)]
            g1 = [pltpu.async_copy(tabl.at[pl.ds(r0 + LB, LB), :],
                                   rows_b.at[pl.ds(0, LB), :], semD),
                  pltpu.async_copy(ball.at[pl.ds(r1 + LB, LB)],
                                   bkt_b.at[pl.ds(0, LB)], semD)]
            for d_ in g0:
                d_.wait()
            accum(bkt_a, rows_a, LB)
            for d_ in g1:
                d_.wait()
            accum(bkt_b, rows_b, LB)
            return 0

        lax.fori_loop(0, nlb // 2, lpair, 0)
        if nlb % 2:
            r0 = lb0 + (nlb - 1) * LB
            r1 = bb0 + (nlb - 1) * LB
            pltpu.sync_copy(tabl.at[pl.ds(r0, LB), :],
                            rows_a.at[pl.ds(0, LB), :])
            pltpu.sync_copy(ball.at[pl.ds(r1, LB)],
                            bkt_a.at[pl.ds(0, LB)])
            accum(bkt_a, rows_a, LB)

    pltpu.sync_copy(acc, grid_sp.at[s])
    plsc.subcore_barrier()
    for si in range(16):
        pltpu.sync_copy(grid_sp.at[si, pl.ds(8 * s, 8), :],
                        rows_a.at[pl.ds(si * 8, 8), :])

    def rz_body(r, _):
        for jj in range(8):
            redout[r, pl.ds(jj * 16, 16)] = _zeros16()
        return 0

    lax.fori_loop(0, 8, rz_body, 0)

    def red_body(si, _):
        for r in range(8):
            for jj in range(8):
                col = jj * 16
                redout[r, pl.ds(col, 16)] = (
                    redout[r, pl.ds(col, 16)]
                    + rows_a[si * 8 + r, pl.ds(col, 16)])
        return 0

    lax.fori_loop(0, 16, red_body, 0)
    pltpu.sync_copy(redout, out_hbm.at[c, pl.ds(8 * s, 8), :])


_pool_kernel = pl.kernel(
    _pool_kernel_body,
    out_type=jax.ShapeDtypeStruct((2, BATCH, D), jnp.float32),
    mesh=_mesh,
    compiler_params=pltpu.CompilerParams(needs_layout_passes=False),
    scratch_types=[
        pltpu.VMEM((BATCH + 8, D), jnp.float32),
        pltpu.VMEM((GSP,), jnp.int32),
        pltpu.VMEM((GSP,), jnp.int32),
        pltpu.VMEM((GSP,), jnp.int32),
        pltpu.VMEM((GSP,), jnp.int32),
        pltpu.VMEM((GSP + 16,), jnp.int32),
        pltpu.VMEM((GSP + 16,), jnp.int32),
        pltpu.VMEM((GSP, D), jnp.float32),
        pltpu.VMEM((GSP, D), jnp.float32),
        pltpu.VMEM((8, D), jnp.float32),
        pltpu.VMEM_SHARED((16, BATCH + 8, D), jnp.float32),
        pltpu.SemaphoreType.DMA,
        pltpu.SemaphoreType.DMA,
        pltpu.SemaphoreType.DMA,
        pltpu.SemaphoreType.DMA,
    ],
)


def _head_body(part_ref, w_ref, b_ref, out_ref):
    pooled = part_ref[0] + part_ref[1]
    logits = jnp.dot(pooled, w_ref[...],
                     preferred_element_type=jnp.float32) + b_ref[...]
    mx = jnp.max(logits, axis=-1, keepdims=True)
    sh = logits - mx
    lse = jnp.log(jnp.sum(jnp.exp(sh), axis=-1, keepdims=True))
    out_ref[...] = sh - lse


def _pad1(a, total, fill):
    pad = total - a.shape[0]
    return jnp.concatenate([a.astype(jnp.int32),
                            fill.astype(jnp.int32)[:pad]])


def kernel(x0, x1, x2, up_index0, up_index1, b1_src, b1_dst, b2_src, b2_dst,
           batch0, batch1, batch2, W, b):
    f32 = jnp.float32
    x0p = jnp.concatenate([x0, jnp.zeros((N0P - N0, D), f32)])
    x1p = jnp.concatenate([x1, jnp.zeros((N1P - N1, D), f32)])
    x2p = jnp.concatenate([x2, jnp.zeros((N2P - N2, D), f32)])

    def spread(n_pad, base, mod):
        return base + (jnp.arange(n_pad, dtype=jnp.int32) % mod)

    s0 = _pad1(up_index0[0], E0P, spread(E0P - 320000, 0, N0))
    d0 = _pad1(up_index0[1], E0P, spread(E0P - 320000, N0, N0P - N0))
    s1 = _pad1(up_index1[0], E1P, spread(E1P - 300000, 0, N1))
    d1 = _pad1(up_index1[1], E1P, spread(E1P - 300000, N1, N1P - N1))
    bs1 = _pad1(b1_src, B1P, spread(B1P - 320000, 0, N0))
    bd1 = _pad1(b1_dst, B1P, spread(B1P - 320000, N1, N1P - N1))
    bs2 = _pad1(b2_src, B2P, spread(B2P - 150000, 0, N1))
    bd2 = _pad1(b2_dst, B2P, spread(B2P - 150000, N2, N2P - N2))

    b0p = _pad1(batch0, N0P, jnp.full((N0P - N0,), BATCH, jnp.int32))
    b1p = _pad1(batch1, N1P, jnp.full((N1P - N1,), BATCH, jnp.int32))
    b2p = _pad1(batch2, N2P, jnp.full((N2P - N2,), BATCH, jnp.int32))

    k1 = _make_scatter_kernel(1, CHUNK0, 2, N0P)
    y0 = k1(x0p, x0p, s0, d0)

    k2 = _make_scatter_kernel(2, CHUNK1, 14, N1P)
    y1 = k2(x1p, x1p, s1, d1, x0, bs1, bd1)

    tabl = jnp.concatenate([y0, y1, x1, x2p])
    ball = jnp.concatenate([b0p, b1p, b2p])
    esall = jnp.concatenate([s0, s1 + OFF_Y1, bs1, bs2 + OFF_Y1,
                             bs2 + OFF_X1])
    edall = jnp.concatenate([d0, d1 + OFF_B1, bd1 + OFF_B1, bd2 + OFF_B2,
                             bd2 + OFF_B2])

    part = _pool_kernel(tabl, esall, edall, ball)

    return pl.pallas_call(
        _head_body,
        out_shape=jax.ShapeDtypeStruct((BATCH, C), f32),
    )(part, W, b.reshape(1, C))
